# baseline (speedup 1.0000x reference)
import jax
import jax.numpy as jnp
from jax import lax
from jax.experimental import pallas as pl
from jax.experimental.pallas import tpu as pltpu
from jax.experimental.pallas import tpu_sc as plsc

VOCAB = 100000
MAXLEN = 200
EMBED = 128
BATCH = 4096

NUM_CORES = 2
NUM_SUBCORES = 16
NW = NUM_CORES * NUM_SUBCORES
SEQ_PER_W = BATCH // NW
ROWS_PER_W = SEQ_PER_W * MAXLEN
LANES = 16
VREGS_PER_ROW = EMBED // LANES
NBUF = 3
SPLIT = 104


def _body(x_hbm, tok_hbm, pos_hbm, out_hbm,
          idx_v, r0, r1, r2, pos_v,
          ga0, ga1, ga2, gb0, gb1, gb2, w0, w1, w2):
    rows = (r0, r1, r2)
    gsa = (ga0, ga1, ga2)
    gsb = (gb0, gb1, gb2)
    ws = (w0, w1, w2)

    wid = lax.axis_index("s") * NUM_CORES + lax.axis_index("c")
    row_base = wid * ROWS_PER_W

    pltpu.sync_copy(x_hbm.at[pl.ds(row_base, ROWS_PER_W)], idx_v)
    pltpu.sync_copy(pos_hbm, pos_v)

    def fire(g, b):
        off = g * MAXLEN
        pltpu.async_copy(
            tok_hbm.at[idx_v.at[pl.ds(off, SPLIT)]],
            rows[b].at[pl.ds(0, SPLIT)], gsa[b])
        pltpu.async_copy(
            tok_hbm.at[idx_v.at[pl.ds(off + SPLIT, MAXLEN - SPLIT)]],
            rows[b].at[pl.ds(SPLIT, MAXLEN - SPLIT)], gsb[b])

    def wait_gather_a(b):
        pltpu.make_async_copy(
            tok_hbm.at[idx_v.at[pl.ds(0, SPLIT)]],
            rows[b].at[pl.ds(0, SPLIT)], gsa[b]).wait()

    def wait_gather_b(b):
        pltpu.make_async_copy(
            tok_hbm.at[idx_v.at[pl.ds(SPLIT, MAXLEN - SPLIT)]],
            rows[b].at[pl.ds(SPLIT, MAXLEN - SPLIT)], gsb[b]).wait()

    def wait_write(b):
        pltpu.make_async_copy(
            rows[b].at[pl.ds(0, SPLIT)],
            out_hbm.at[pl.ds(row_base, SPLIT)], ws[b]).wait()
        pltpu.make_async_copy(
            rows[b].at[pl.ds(SPLIT, MAXLEN - SPLIT)],
            out_hbm.at[pl.ds(row_base, MAXLEN - SPLIT)], ws[b]).wait()

    fire(0, 0)

    def outer(k, carry):
        for b in range(NBUF):
            g = k * NBUF + b
            nb = (b + 1) % NBUF

            @pl.when(g < SEQ_PER_W)
            def _chunk():
                @pl.when(g + 1 < SEQ_PER_W)
                def _prefetch():
                    @pl.when(g + 1 >= NBUF)
                    def _drain():
                        wait_write(nb)
                    fire(g + 1, nb)

                def add_row(i, acc):
                    for j in range(VREGS_PER_ROW):
                        sl = pl.ds(j * LANES, LANES)
                        rows[b][i, sl] = rows[b][i, sl] + pos_v[i, sl]
                    return acc

                out_off = row_base + g * MAXLEN

                wait_gather_a(b)
                lax.fori_loop(0, SPLIT, add_row, 0)
                pltpu.async_copy(
                    rows[b].at[pl.ds(0, SPLIT)],
                    out_hbm.at[pl.ds(out_off, SPLIT)], ws[b])

                wait_gather_b(b)
                lax.fori_loop(SPLIT, MAXLEN, add_row, 0)
                pltpu.async_copy(
                    rows[b].at[pl.ds(SPLIT, MAXLEN - SPLIT)],
                    out_hbm.at[pl.ds(out_off + SPLIT, MAXLEN - SPLIT)], ws[b])
        return carry

    lax.fori_loop(0, (SEQ_PER_W + NBUF - 1) // NBUF, outer, 0)

    for b in range(NBUF):
        wait_write(b)


@jax.jit
def kernel(x, token_table, pos_table):
    x_flat = x.reshape(-1).astype(jnp.int32)
    mesh = plsc.VectorSubcoreMesh(core_axis_name="c", subcore_axis_name="s")
    out = pl.kernel(
        _body,
        mesh=mesh,
        out_type=jax.ShapeDtypeStruct((BATCH * MAXLEN, EMBED), jnp.float32),
        scratch_types=(
            [pltpu.VMEM((ROWS_PER_W,), jnp.int32)]
            + [pltpu.VMEM((MAXLEN, EMBED), jnp.float32) for _ in range(NBUF)]
            + [pltpu.VMEM((MAXLEN, EMBED), jnp.float32)]
            + [pltpu.SemaphoreType.DMA for _ in range(3 * NBUF)]
        ),
    )(x_flat, token_table, pos_table)
    return out.reshape(BATCH, MAXLEN, EMBED)
